# Initial kernel scaffold; baseline (speedup 1.0000x reference)
import jax
import jax.numpy as jnp
from jax.experimental import pallas as pl


def kernel(x):
    raise NotImplementedError("write your pallas kernel here")


# speedup vs baseline: 1381.5898x; 1381.5898x over previous
import functools

import jax
import jax.numpy as jnp
import numpy as np
from jax.experimental import pallas as pl
from jax.experimental.pallas import tpu as pltpu

_IN = 64
_OUT = 96
_C = 8
_BZ = 24


def _interp_matrix() -> np.ndarray:
    loc = np.linspace(0.0, _IN - 1.0, _OUT)
    k0 = np.clip(np.floor(loc), 0, _IN - 1).astype(np.int64)
    k1 = np.clip(k0 + 1, 0, _IN - 1)
    w0 = k1.astype(np.float64) - loc
    w1 = 1.0 - w0
    W = np.zeros((_OUT, _IN), dtype=np.float64)
    W[np.arange(_OUT), k0] += w0
    W[np.arange(_OUT), k1] += w1
    return W.astype(np.float32)


def _resize_body(wz_ref, w_ref, mk_ref, x_ref, o_ref):
    A = x_ref[0]
    t1 = jnp.dot(wz_ref[...], A, preferred_element_type=jnp.float32)
    t13 = t1.reshape(_BZ, _IN, _IN * _C)
    w = w_ref[...]
    mk = mk_ref[...]
    for z in range(_BZ):
        t2 = jnp.dot(w, t13[z], preferred_element_type=jnp.float32)
        o_ref[0, z] = jnp.dot(t2, mk, preferred_element_type=jnp.float32)


@jax.jit
def kernel(x):
    B = x.shape[0]
    W = jnp.asarray(_interp_matrix())
    MK = np.einsum("jk,ce->kcje", _interp_matrix(), np.eye(_C, dtype=np.float32))
    MK = jnp.asarray(MK.reshape(_IN * _C, _OUT * _C))

    x2 = x.reshape(B, _IN, _IN * _IN * _C)
    grid = (B, _OUT // _BZ)
    out = pl.pallas_call(
        _resize_body,
        grid=grid,
        in_specs=[
            pl.BlockSpec((_BZ, _IN), lambda b, q: (q, 0)),
            pl.BlockSpec((_OUT, _IN), lambda b, q: (0, 0)),
            pl.BlockSpec((_IN * _C, _OUT * _C), lambda b, q: (0, 0)),
            pl.BlockSpec((1, _IN, _IN * _IN * _C), lambda b, q: (b, 0, 0)),
        ],
        out_specs=pl.BlockSpec((1, _BZ, _OUT, _OUT * _C), lambda b, q: (b, q, 0, 0)),
        out_shape=jax.ShapeDtypeStruct((B, _OUT, _OUT, _OUT * _C), jnp.float32),
    )(W, W, MK, x2)
    return out.reshape(B, _OUT, _OUT, _OUT, _C)
